# Initial kernel scaffold; baseline (speedup 1.0000x reference)
import functools

import jax
import jax.numpy as jnp
from jax import lax
from jax.experimental import pallas as pl
from jax.experimental.pallas import tpu as pltpu
from jax.experimental.pallas import tpu_sc as plsc

NC = 2
NS = 16
NW = NC * NS
L = 16

NNODES = 10000
NEDGES = 320000
D = 128
DP = 144
BLK = 80
BLKS = NEDGES // NW // BLK
ROWS_PER_TILE = NNODES // NS
HP = 10240
HSL = HP // NS


def _sc_hist(row3):
    @functools.partial(
        pl.kernel,
        out_type=jax.ShapeDtypeStruct((NC, HP), jnp.float32),
        mesh=plsc.VectorSubcoreMesh(core_axis_name="c", subcore_axis_name="s"),
        scratch_types=[
            pltpu.VMEM((BLKS, BLK), jnp.int32),
            pltpu.VMEM((BLK,), jnp.float32),
            pltpu.VMEM((HSL,), jnp.float32),
            pltpu.VMEM_SHARED((HP,), jnp.float32),
        ],
    )
    def k(row_hbm, out_hbm, idx_v, ones_v, z_v, hist_sh):
        c = lax.axis_index("c")
        s = lax.axis_index("s")
        w = c * NS + s

        @pl.loop(0, BLK, step=L)
        def _init_ones(i):
            ones_v[pl.ds(i, L)] = jnp.ones((L,), jnp.float32)

        @pl.loop(0, HSL, step=L)
        def _init_zeros(i):
            z_v[pl.ds(i, L)] = jnp.zeros((L,), jnp.float32)

        pltpu.sync_copy(z_v, hist_sh.at[pl.ds(s * HSL, HSL)])
        plsc.subcore_barrier()

        pltpu.sync_copy(row_hbm.at[w], idx_v)

        @pl.loop(0, BLKS)
        def _accum(j):
            pltpu.sync_copy(ones_v, hist_sh.at[idx_v.at[j]], add=True)

        plsc.subcore_barrier()
        pltpu.sync_copy(hist_sh.at[pl.ds(s * HSL, HSL)],
                        out_hbm.at[c, pl.ds(s * HSL, HSL)])

    return k(row3)


def _sc_agg(row3, col3, xs_ext):
    @functools.partial(
        pl.kernel,
        out_type=jax.ShapeDtypeStruct((NC, NNODES, DP), jnp.float32),
        mesh=plsc.VectorSubcoreMesh(core_axis_name="c", subcore_axis_name="s"),
        scratch_types=[
            pltpu.VMEM((BLKS, BLK), jnp.int32),
            pltpu.VMEM((BLKS, BLK), jnp.int32),
            pltpu.VMEM((BLK, DP), jnp.float32),
            pltpu.VMEM((125, DP), jnp.float32),
            pltpu.VMEM_SHARED((NNODES, DP), jnp.float32),
        ],
    )
    def k(row_hbm, col_hbm, xs_hbm, out_hbm, ri_v, ci_v, val_v, z_v, acc_sh):
        c = lax.axis_index("c")
        s = lax.axis_index("s")
        w = c * NS + s

        @pl.loop(0, 125)
        def _zrow(i):
            @pl.loop(0, DP, step=L)
            def _zcol(j):
                z_v[i, pl.ds(j, L)] = jnp.zeros((L,), jnp.float32)

        @pl.loop(0, ROWS_PER_TILE, step=125)
        def _zacc(r):
            pltpu.sync_copy(z_v, acc_sh.at[pl.ds(s * ROWS_PER_TILE + r, 125)])

        plsc.subcore_barrier()

        pltpu.sync_copy(row_hbm.at[w], ri_v)
        pltpu.sync_copy(col_hbm.at[w], ci_v)

        @pl.loop(0, BLKS)
        def _edges(j):
            pltpu.sync_copy(xs_hbm.at[ri_v.at[j]], val_v)
            pltpu.sync_copy(val_v, acc_sh.at[ci_v.at[j]], add=True)

        plsc.subcore_barrier()

        @pl.loop(0, ROWS_PER_TILE, step=125)
        def _out(r):
            pltpu.sync_copy(acc_sh.at[pl.ds(s * ROWS_PER_TILE + r, 125)],
                            out_hbm.at[c, pl.ds(s * ROWS_PER_TILE + r, 125)])

    return k(row3, col3, xs_ext)


_BM = 2000


def _tc_prep(hist_t, x):
    def body(h_ref, x_ref, o_ref):
        h = h_ref[...]
        deg = h[:, 0:1] + h[:, 1:2] + 1.0
        dis = lax.rsqrt(deg)
        o_ref[...] = jnp.concatenate(
            [dis * x_ref[...], dis, jnp.zeros((_BM, DP - D - 1), jnp.float32)],
            axis=1)

    return pl.pallas_call(
        body,
        grid=(NNODES // _BM,),
        in_specs=[
            pl.BlockSpec((_BM, 2), lambda i: (i, 0)),
            pl.BlockSpec((_BM, D), lambda i: (i, 0)),
        ],
        out_specs=pl.BlockSpec((_BM, DP), lambda i: (i, 0)),
        out_shape=jax.ShapeDtypeStruct((NNODES, DP), jnp.float32),
    )(hist_t, x)


def _tc_dense(x, xs_ext, parts, W_low, b_low, W_high, b_high, W_id, b_id,
              W_gate, b_gate):
    def body(x_ref, xe_ref, p_ref, wl, bl, wh, bh, wi, bi, wg, bg, o_ref):
        p = p_ref[...]
        xe = xe_ref[...]
        AGs = xe[:, D:D + 1] * (p[0] + p[1] + xe)
        AG = AGs[:, :D]
        s = AGs[:, D:D + 1]
        xv = x_ref[...]
        agg_low = jnp.dot(AG, wl[...],
                          preferred_element_type=jnp.float32) + s * bl[...]
        agg_high = (jnp.dot(xv - AG, wh[...],
                            preferred_element_type=jnp.float32)
                    + (1.0 - s) * bh[...])
        x_id = jnp.dot(xv, wi[...], preferred_element_type=jnp.float32) + bi[...]
        logits = jnp.dot(xv, wg[...],
                         preferred_element_type=jnp.float32) + bg[...]
        m = jnp.max(logits, axis=1, keepdims=True)
        e = jnp.exp(logits - m)
        g = e / jnp.sum(e, axis=1, keepdims=True)
        o_ref[...] = (g[:, 0:1] * agg_low + g[:, 1:2] * agg_high
                      + g[:, 2:3] * x_id)

    def full(shape):
        return pl.BlockSpec(shape, lambda i: tuple(0 for _ in shape))

    return pl.pallas_call(
        body,
        grid=(NNODES // _BM,),
        in_specs=[
            pl.BlockSpec((_BM, D), lambda i: (i, 0)),
            pl.BlockSpec((_BM, DP), lambda i: (i, 0)),
            pl.BlockSpec((NC, _BM, DP), lambda i: (0, i, 0)),
            full((D, D)), full((1, D)),
            full((D, D)), full((1, D)),
            full((D, D)), full((1, D)),
            full((D, 3)), full((1, 3)),
        ],
        out_specs=pl.BlockSpec((_BM, D), lambda i: (i, 0)),
        out_shape=jax.ShapeDtypeStruct((NNODES, D), jnp.float32),
    )(x, xs_ext, parts, W_low, b_low.reshape(1, D), W_high,
      b_high.reshape(1, D), W_id, b_id.reshape(1, D), W_gate,
      b_gate.reshape(1, 3))


def kernel(x, edge_index, W_low, b_low, W_high, b_high, W_id, b_id, W_gate,
           b_gate):
    row3 = edge_index[0].reshape(NW, BLKS, BLK)
    col3 = edge_index[1].reshape(NW, BLKS, BLK)
    hist = _sc_hist(row3)
    hist_t = hist[:, :NNODES].T
    xs_ext = _tc_prep(hist_t, x)
    parts = _sc_agg(row3, col3, xs_ext)
    return _tc_dense(x, xs_ext, parts, W_low, b_low, W_high, b_high, W_id,
                     b_id, W_gate, b_gate)


# speedup vs baseline: 30.0184x; 30.0184x over previous
import functools

import jax
import jax.numpy as jnp
from jax import lax
from jax.experimental import pallas as pl
from jax.experimental.pallas import tpu as pltpu
from jax.experimental.pallas import tpu_sc as plsc

NC = 2
NS = 16
NW = NC * NS
L = 16

NNODES = 10000
NEDGES = 320000
D = 128
DP = 144
BLK = 80
BLKS = NEDGES // NW // BLK
APAD = 10240
ROWS_PER_TILE = APAD // NS
CB = 25
HP = 10240
HSL = HP // NS


def _sc_hist(row3):
    @functools.partial(
        pl.kernel,
        out_type=jax.ShapeDtypeStruct((NC, HP), jnp.float32),
        mesh=plsc.VectorSubcoreMesh(core_axis_name="c", subcore_axis_name="s"),
        compiler_params=pltpu.CompilerParams(use_tc_tiling_on_sc=False),
        scratch_types=[
            pltpu.VMEM((BLKS, BLK), jnp.int32),
            pltpu.VMEM((BLK,), jnp.float32),
            pltpu.VMEM((HSL,), jnp.float32),
            pltpu.VMEM_SHARED((HP,), jnp.float32),
        ],
    )
    def k(row_hbm, out_hbm, idx_v, ones_v, z_v, hist_sh):
        c = lax.axis_index("c")
        s = lax.axis_index("s")
        w = c * NS + s

        @pl.loop(0, BLK, step=L)
        def _init_ones(i):
            ones_v[pl.ds(i, L)] = jnp.ones((L,), jnp.float32)

        @pl.loop(0, HSL, step=L)
        def _init_zeros(i):
            z_v[pl.ds(i, L)] = jnp.zeros((L,), jnp.float32)

        pltpu.sync_copy(z_v, hist_sh.at[pl.ds(s * HSL, HSL)])
        plsc.subcore_barrier()

        pltpu.sync_copy(row_hbm.at[w], idx_v)

        @pl.loop(0, BLKS)
        def _accum(j):
            pltpu.sync_copy(ones_v, hist_sh.at[idx_v.at[j]], add=True)

        plsc.subcore_barrier()
        pltpu.sync_copy(hist_sh.at[pl.ds(s * HSL, HSL)],
                        out_hbm.at[c, pl.ds(s * HSL, HSL)])

    return k(row3)


def _sc_agg(row3, col3, xs_ext):
    @functools.partial(
        pl.kernel,
        out_type=jax.ShapeDtypeStruct((NC, APAD, DP), jnp.float32),
        mesh=plsc.VectorSubcoreMesh(core_axis_name="c", subcore_axis_name="s"),
        compiler_params=pltpu.CompilerParams(use_tc_tiling_on_sc=False),
        scratch_types=[
            pltpu.VMEM((CB, BLK), jnp.int32),
            pltpu.VMEM((CB, BLK), jnp.int32),
            pltpu.VMEM((BLK, DP), jnp.float32),
            pltpu.VMEM_SHARED((APAD, DP), jnp.float32),
        ],
    )
    def k(row_hbm, col_hbm, xs_hbm, out_hbm, ri_v, ci_v, val_v, acc_sh):
        c = lax.axis_index("c")
        s = lax.axis_index("s")
        w = c * NS + s

        @pl.loop(0, BLK)
        def _zrow(i):
            @pl.loop(0, DP, step=L)
            def _zcol(j):
                val_v[i, pl.ds(j, L)] = jnp.zeros((L,), jnp.float32)

        @pl.loop(0, ROWS_PER_TILE, step=BLK)
        def _zacc(r):
            pltpu.sync_copy(val_v, acc_sh.at[pl.ds(s * ROWS_PER_TILE + r, BLK)])

        plsc.subcore_barrier()

        @pl.loop(0, BLKS, step=CB)
        def _chunk(jc):
            pltpu.sync_copy(row_hbm.at[w, pl.ds(jc, CB)], ri_v)
            pltpu.sync_copy(col_hbm.at[w, pl.ds(jc, CB)], ci_v)

            @pl.loop(0, CB)
            def _edges(j):
                pltpu.sync_copy(xs_hbm.at[ri_v.at[j]], val_v)
                pltpu.sync_copy(val_v, acc_sh.at[ci_v.at[j]], add=True)

        plsc.subcore_barrier()

        pltpu.sync_copy(acc_sh.at[pl.ds(s * ROWS_PER_TILE, ROWS_PER_TILE)],
                        out_hbm.at[c, pl.ds(s * ROWS_PER_TILE, ROWS_PER_TILE)])

    return k(row3, col3, xs_ext)


_BM = 2000


def _tc_prep(hist_t, x):
    def body(h_ref, x_ref, o_ref):
        h = h_ref[...]
        deg = h[:, 0:1] + h[:, 1:2] + 1.0
        dis = lax.rsqrt(deg)
        o_ref[...] = jnp.concatenate(
            [dis * x_ref[...], dis, jnp.zeros((_BM, DP - D - 1), jnp.float32)],
            axis=1)

    return pl.pallas_call(
        body,
        grid=(NNODES // _BM,),
        in_specs=[
            pl.BlockSpec((_BM, 2), lambda i: (i, 0)),
            pl.BlockSpec((_BM, D), lambda i: (i, 0)),
        ],
        out_specs=pl.BlockSpec((_BM, DP), lambda i: (i, 0)),
        out_shape=jax.ShapeDtypeStruct((NNODES, DP), jnp.float32),
    )(hist_t, x)


def _tc_dense(x, xs_ext, parts, W_low, b_low, W_high, b_high, W_id, b_id,
              W_gate, b_gate):
    def body(x_ref, xe_ref, p_ref, wl, bl, wh, bh, wi, bi, wg, bg, o_ref):
        p = p_ref[...]
        xe = xe_ref[...]
        AGs = xe[:, D:D + 1] * (p[0] + p[1] + xe)
        AG = AGs[:, :D]
        s = AGs[:, D:D + 1]
        xv = x_ref[...]
        agg_low = jnp.dot(AG, wl[...],
                          preferred_element_type=jnp.float32) + s * bl[...]
        agg_high = (jnp.dot(xv - AG, wh[...],
                            preferred_element_type=jnp.float32)
                    + (1.0 - s) * bh[...])
        x_id = jnp.dot(xv, wi[...], preferred_element_type=jnp.float32) + bi[...]
        logits = jnp.dot(xv, wg[...],
                         preferred_element_type=jnp.float32) + bg[...]
        m = jnp.max(logits, axis=1, keepdims=True)
        e = jnp.exp(logits - m)
        g = e / jnp.sum(e, axis=1, keepdims=True)
        o_ref[...] = (g[:, 0:1] * agg_low + g[:, 1:2] * agg_high
                      + g[:, 2:3] * x_id)

    def full(shape):
        return pl.BlockSpec(shape, lambda i: tuple(0 for _ in shape))

    return pl.pallas_call(
        body,
        grid=(NNODES // _BM,),
        in_specs=[
            pl.BlockSpec((_BM, D), lambda i: (i, 0)),
            pl.BlockSpec((_BM, DP), lambda i: (i, 0)),
            pl.BlockSpec((NC, _BM, DP), lambda i: (0, i, 0)),
            full((D, D)), full((1, D)),
            full((D, D)), full((1, D)),
            full((D, D)), full((1, D)),
            full((D, 3)), full((1, 3)),
        ],
        out_specs=pl.BlockSpec((_BM, D), lambda i: (i, 0)),
        out_shape=jax.ShapeDtypeStruct((NNODES, D), jnp.float32),
    )(x, xs_ext, parts, W_low, b_low.reshape(1, D), W_high,
      b_high.reshape(1, D), W_id, b_id.reshape(1, D), W_gate,
      b_gate.reshape(1, 3))


def kernel(x, edge_index, W_low, b_low, W_high, b_high, W_id, b_id, W_gate,
           b_gate):
    row3 = edge_index[0].reshape(NW, BLKS, BLK)
    col3 = edge_index[1].reshape(NW, BLKS, BLK)
    hist = _sc_hist(row3)
    hist_t = hist[:, :NNODES].T
    xs_ext = _tc_prep(hist_t, x)
    parts = _sc_agg(row3, col3, xs_ext)
    return _tc_dense(x, xs_ext, parts, W_low, b_low, W_high, b_high, W_id,
                     b_id, W_gate, b_gate)


# speedup vs baseline: 41.7084x; 1.3894x over previous
import functools

import jax
import jax.numpy as jnp
from jax import lax
from jax.experimental import pallas as pl
from jax.experimental.pallas import tpu as pltpu
from jax.experimental.pallas import tpu_sc as plsc

NC = 2
NS = 16
NW = NC * NS
L = 16

NNODES = 10000
NEDGES = 320000
D = 128
DP = 144
BLK = 80
BLKS = NEDGES // NW // BLK
ABLK = 100
ABLKS = NEDGES // NW // ABLK
APAD = 10240
ROWS_PER_TILE = APAD // NS
CB = 20
NCHUNK = ABLKS // CB
ZR = 80
HP = 10240
HSL = HP // NS


def _sc_hist(row3):
    @functools.partial(
        pl.kernel,
        out_type=jax.ShapeDtypeStruct((NC, HP), jnp.float32),
        mesh=plsc.VectorSubcoreMesh(core_axis_name="c", subcore_axis_name="s"),
        compiler_params=pltpu.CompilerParams(use_tc_tiling_on_sc=False),
        scratch_types=[
            pltpu.VMEM((BLKS, BLK), jnp.int32),
            pltpu.VMEM((BLK,), jnp.float32),
            pltpu.VMEM((HSL,), jnp.float32),
            pltpu.VMEM_SHARED((HP,), jnp.float32),
        ],
    )
    def k(row_hbm, out_hbm, idx_v, ones_v, z_v, hist_sh):
        c = lax.axis_index("c")
        s = lax.axis_index("s")
        w = c * NS + s

        @pl.loop(0, BLK, step=L)
        def _init_ones(i):
            ones_v[pl.ds(i, L)] = jnp.ones((L,), jnp.float32)

        @pl.loop(0, HSL, step=L)
        def _init_zeros(i):
            z_v[pl.ds(i, L)] = jnp.zeros((L,), jnp.float32)

        pltpu.sync_copy(z_v, hist_sh.at[pl.ds(s * HSL, HSL)])
        plsc.subcore_barrier()

        pltpu.sync_copy(row_hbm.at[w], idx_v)

        @pl.loop(0, BLKS)
        def _accum(j):
            pltpu.sync_copy(ones_v, hist_sh.at[idx_v.at[j]], add=True)

        plsc.subcore_barrier()
        pltpu.sync_copy(hist_sh.at[pl.ds(s * HSL, HSL)],
                        out_hbm.at[c, pl.ds(s * HSL, HSL)])

    return k(row3)


def _sc_agg(row3, col3, xs_ext):
    @functools.partial(
        pl.kernel,
        out_type=jax.ShapeDtypeStruct((NC, APAD, DP), jnp.float32),
        mesh=plsc.VectorSubcoreMesh(core_axis_name="c", subcore_axis_name="s"),
        compiler_params=pltpu.CompilerParams(use_tc_tiling_on_sc=False),
        scratch_types=[
            pltpu.VMEM((CB, ABLK), jnp.int32),
            pltpu.VMEM((CB, ABLK), jnp.int32),
            pltpu.VMEM((ABLK, DP), jnp.float32),
            pltpu.VMEM((ABLK, DP), jnp.float32),
            pltpu.SemaphoreType.DMA,
            pltpu.SemaphoreType.DMA,
            pltpu.SemaphoreType.DMA,
            pltpu.SemaphoreType.DMA,
            pltpu.VMEM_SHARED((APAD, DP), jnp.float32),
        ],
    )
    def k(row_hbm, col_hbm, xs_hbm, out_hbm, ri_v, ci_v, va, vb,
          sga, sgb, ssa, ssb, acc_sh):
        c = lax.axis_index("c")
        s = lax.axis_index("s")
        w = c * NS + s

        @pl.loop(0, ZR)
        def _zrow(i):
            @pl.loop(0, DP, step=L)
            def _zcol(j):
                va[i, pl.ds(j, L)] = jnp.zeros((L,), jnp.float32)

        @pl.loop(0, ROWS_PER_TILE, step=ZR)
        def _zacc(r):
            pltpu.sync_copy(va.at[pl.ds(0, ZR)],
                            acc_sh.at[pl.ds(s * ROWS_PER_TILE + r, ZR)])

        plsc.subcore_barrier()

        @pl.loop(0, NCHUNK)
        def _chunk(ch):
            jc = ch * CB
            pltpu.sync_copy(row_hbm.at[w, pl.ds(jc, CB)], ri_v)
            pltpu.sync_copy(col_hbm.at[w, pl.ds(jc, CB)], ci_v)
            pltpu.async_copy(xs_hbm.at[ri_v.at[0]], va, sga)
            pltpu.async_copy(xs_hbm.at[ri_v.at[1]], vb, sgb)

            @pl.loop(0, CB - 2, step=2)
            def _pair(j):
                pltpu.make_async_copy(xs_hbm.at[ri_v.at[j]], va, sga).wait()
                pltpu.async_copy(va, acc_sh.at[ci_v.at[j]], ssa, add=True)
                pltpu.make_async_copy(va, acc_sh.at[ci_v.at[j]], ssa).wait()
                pltpu.async_copy(xs_hbm.at[ri_v.at[j + 2]], va, sga)
                pltpu.make_async_copy(xs_hbm.at[ri_v.at[j + 1]], vb, sgb).wait()
                pltpu.async_copy(vb, acc_sh.at[ci_v.at[j + 1]], ssb, add=True)
                pltpu.make_async_copy(vb, acc_sh.at[ci_v.at[j + 1]], ssb).wait()
                pltpu.async_copy(xs_hbm.at[ri_v.at[j + 3]], vb, sgb)

            pltpu.make_async_copy(xs_hbm.at[ri_v.at[CB - 2]], va, sga).wait()
            pltpu.async_copy(va, acc_sh.at[ci_v.at[CB - 2]], ssa, add=True)
            pltpu.make_async_copy(xs_hbm.at[ri_v.at[CB - 1]], vb, sgb).wait()
            pltpu.async_copy(vb, acc_sh.at[ci_v.at[CB - 1]], ssb, add=True)
            pltpu.make_async_copy(va, acc_sh.at[ci_v.at[CB - 2]], ssa).wait()
            pltpu.make_async_copy(vb, acc_sh.at[ci_v.at[CB - 1]], ssb).wait()

        plsc.subcore_barrier()

        pltpu.sync_copy(acc_sh.at[pl.ds(s * ROWS_PER_TILE, ROWS_PER_TILE)],
                        out_hbm.at[c, pl.ds(s * ROWS_PER_TILE, ROWS_PER_TILE)])

    return k(row3, col3, xs_ext)


_BM = 2000


def _tc_prep(hist_t, x):
    def body(h_ref, x_ref, o_ref):
        h = h_ref[...]
        deg = h[:, 0:1] + h[:, 1:2] + 1.0
        dis = lax.rsqrt(deg)
        o_ref[...] = jnp.concatenate(
            [dis * x_ref[...], dis, jnp.zeros((_BM, DP - D - 1), jnp.float32)],
            axis=1)

    return pl.pallas_call(
        body,
        grid=(NNODES // _BM,),
        in_specs=[
            pl.BlockSpec((_BM, 2), lambda i: (i, 0)),
            pl.BlockSpec((_BM, D), lambda i: (i, 0)),
        ],
        out_specs=pl.BlockSpec((_BM, DP), lambda i: (i, 0)),
        out_shape=jax.ShapeDtypeStruct((NNODES, DP), jnp.float32),
    )(hist_t, x)


def _tc_dense(x, xs_ext, parts, W_low, b_low, W_high, b_high, W_id, b_id,
              W_gate, b_gate):
    def body(x_ref, xe_ref, p_ref, wl, bl, wh, bh, wi, bi, wg, bg, o_ref):
        p = p_ref[...]
        xe = xe_ref[...]
        AGs = xe[:, D:D + 1] * (p[0] + p[1] + xe)
        AG = AGs[:, :D]
        s = AGs[:, D:D + 1]
        xv = x_ref[...]
        agg_low = jnp.dot(AG, wl[...],
                          preferred_element_type=jnp.float32) + s * bl[...]
        agg_high = (jnp.dot(xv - AG, wh[...],
                            preferred_element_type=jnp.float32)
                    + (1.0 - s) * bh[...])
        x_id = jnp.dot(xv, wi[...], preferred_element_type=jnp.float32) + bi[...]
        logits = jnp.dot(xv, wg[...],
                         preferred_element_type=jnp.float32) + bg[...]
        m = jnp.max(logits, axis=1, keepdims=True)
        e = jnp.exp(logits - m)
        g = e / jnp.sum(e, axis=1, keepdims=True)
        o_ref[...] = (g[:, 0:1] * agg_low + g[:, 1:2] * agg_high
                      + g[:, 2:3] * x_id)

    def full(shape):
        return pl.BlockSpec(shape, lambda i: tuple(0 for _ in shape))

    return pl.pallas_call(
        body,
        grid=(NNODES // _BM,),
        in_specs=[
            pl.BlockSpec((_BM, D), lambda i: (i, 0)),
            pl.BlockSpec((_BM, DP), lambda i: (i, 0)),
            pl.BlockSpec((NC, _BM, DP), lambda i: (0, i, 0)),
            full((D, D)), full((1, D)),
            full((D, D)), full((1, D)),
            full((D, D)), full((1, D)),
            full((D, 3)), full((1, 3)),
        ],
        out_specs=pl.BlockSpec((_BM, D), lambda i: (i, 0)),
        out_shape=jax.ShapeDtypeStruct((NNODES, D), jnp.float32),
    )(x, xs_ext, parts, W_low, b_low.reshape(1, D), W_high,
      b_high.reshape(1, D), W_id, b_id.reshape(1, D), W_gate,
      b_gate.reshape(1, 3))


def kernel(x, edge_index, W_low, b_low, W_high, b_high, W_id, b_id, W_gate,
           b_gate):
    row_h = edge_index[0].reshape(NW, BLKS, BLK)
    row3 = edge_index[0].reshape(NW, ABLKS, ABLK)
    col3 = edge_index[1].reshape(NW, ABLKS, ABLK)
    hist = _sc_hist(row_h)
    hist_t = hist[:, :NNODES].T
    xs_ext = _tc_prep(hist_t, x)
    parts = _sc_agg(row3, col3, xs_ext)
    return _tc_dense(x, xs_ext, parts, W_low, b_low, W_high, b_high, W_id,
                     b_id, W_gate, b_gate)


# speedup vs baseline: 41.8587x; 1.0036x over previous
import functools

import jax
import jax.numpy as jnp
from jax import lax
from jax.experimental import pallas as pl
from jax.experimental.pallas import tpu as pltpu
from jax.experimental.pallas import tpu_sc as plsc

NC = 2
NS = 16
NW = NC * NS
L = 16

NNODES = 10000
NEDGES = 320000
D = 128
DP = 144
BLK = 80
BLKS = NEDGES // NW // BLK
ABLK = 100
ABLKS = NEDGES // NW // ABLK
APAD = 10240
ROWS_PER_TILE = APAD // NS
CB = 20
NCHUNK = ABLKS // CB
ZR = 80
HP = 10240
HSL = HP // NS


def _sc_hist(ei3):
    @functools.partial(
        pl.kernel,
        out_type=jax.ShapeDtypeStruct((NC, HP), jnp.float32),
        mesh=plsc.VectorSubcoreMesh(core_axis_name="c", subcore_axis_name="s"),
        compiler_params=pltpu.CompilerParams(use_tc_tiling_on_sc=False),
        scratch_types=[
            pltpu.VMEM((CB, ABLK), jnp.int32),
            pltpu.VMEM((112,), jnp.float32),
            pltpu.VMEM((HSL,), jnp.float32),
            pltpu.VMEM_SHARED((HP,), jnp.float32),
        ],
    )
    def k(ei_hbm, out_hbm, idx_v, ones_v, z_v, hist_sh):
        c = lax.axis_index("c")
        s = lax.axis_index("s")
        w = c * NS + s

        @pl.loop(0, 112, step=L)
        def _init_ones(i):
            ones_v[pl.ds(i, L)] = jnp.ones((L,), jnp.float32)

        @pl.loop(0, HSL, step=L)
        def _init_zeros(i):
            z_v[pl.ds(i, L)] = jnp.zeros((L,), jnp.float32)

        pltpu.sync_copy(z_v, hist_sh.at[pl.ds(s * HSL, HSL)])
        plsc.subcore_barrier()

        @pl.loop(0, NCHUNK)
        def _chunk(ch):
            pltpu.sync_copy(ei_hbm.at[0, pl.ds(w * ABLKS + ch * CB, CB)],
                            idx_v)

            @pl.loop(0, CB)
            def _accum(j):
                pltpu.sync_copy(ones_v.at[pl.ds(0, ABLK)],
                                hist_sh.at[idx_v.at[j]], add=True)

        plsc.subcore_barrier()
        pltpu.sync_copy(hist_sh.at[pl.ds(s * HSL, HSL)],
                        out_hbm.at[c, pl.ds(s * HSL, HSL)])

    return k(ei3)


def _sc_agg(ei3, xs_ext):
    @functools.partial(
        pl.kernel,
        out_type=jax.ShapeDtypeStruct((NC, APAD, DP), jnp.float32),
        mesh=plsc.VectorSubcoreMesh(core_axis_name="c", subcore_axis_name="s"),
        compiler_params=pltpu.CompilerParams(use_tc_tiling_on_sc=False),
        scratch_types=[
            pltpu.VMEM((CB, ABLK), jnp.int32),
            pltpu.VMEM((CB, ABLK), jnp.int32),
            pltpu.VMEM((ABLK, DP), jnp.float32),
            pltpu.VMEM((ABLK, DP), jnp.float32),
            pltpu.SemaphoreType.DMA,
            pltpu.SemaphoreType.DMA,
            pltpu.SemaphoreType.DMA,
            pltpu.SemaphoreType.DMA,
            pltpu.VMEM_SHARED((APAD, DP), jnp.float32),
        ],
    )
    def k(ei_hbm, xs_hbm, out_hbm, ri_v, ci_v, va, vb,
          sga, sgb, ssa, ssb, acc_sh):
        c = lax.axis_index("c")
        s = lax.axis_index("s")
        w = c * NS + s

        @pl.loop(0, ZR)
        def _zrow(i):
            @pl.loop(0, DP, step=L)
            def _zcol(j):
                va[i, pl.ds(j, L)] = jnp.zeros((L,), jnp.float32)

        @pl.loop(0, ROWS_PER_TILE, step=ZR)
        def _zacc(r):
            pltpu.sync_copy(va.at[pl.ds(0, ZR)],
                            acc_sh.at[pl.ds(s * ROWS_PER_TILE + r, ZR)])

        plsc.subcore_barrier()

        @pl.loop(0, NCHUNK)
        def _chunk(ch):
            jc = w * ABLKS + ch * CB
            pltpu.sync_copy(ei_hbm.at[0, pl.ds(jc, CB)], ri_v)
            pltpu.sync_copy(ei_hbm.at[1, pl.ds(jc, CB)], ci_v)
            pltpu.async_copy(xs_hbm.at[ri_v.at[0]], va, sga)
            pltpu.async_copy(xs_hbm.at[ri_v.at[1]], vb, sgb)

            @pl.loop(0, CB - 2, step=2)
            def _pair(j):
                pltpu.make_async_copy(xs_hbm.at[ri_v.at[j]], va, sga).wait()
                pltpu.async_copy(va, acc_sh.at[ci_v.at[j]], ssa, add=True)
                pltpu.make_async_copy(va, acc_sh.at[ci_v.at[j]], ssa).wait()
                pltpu.async_copy(xs_hbm.at[ri_v.at[j + 2]], va, sga)
                pltpu.make_async_copy(xs_hbm.at[ri_v.at[j + 1]], vb, sgb).wait()
                pltpu.async_copy(vb, acc_sh.at[ci_v.at[j + 1]], ssb, add=True)
                pltpu.make_async_copy(vb, acc_sh.at[ci_v.at[j + 1]], ssb).wait()
                pltpu.async_copy(xs_hbm.at[ri_v.at[j + 3]], vb, sgb)

            pltpu.make_async_copy(xs_hbm.at[ri_v.at[CB - 2]], va, sga).wait()
            pltpu.async_copy(va, acc_sh.at[ci_v.at[CB - 2]], ssa, add=True)
            pltpu.make_async_copy(xs_hbm.at[ri_v.at[CB - 1]], vb, sgb).wait()
            pltpu.async_copy(vb, acc_sh.at[ci_v.at[CB - 1]], ssb, add=True)
            pltpu.make_async_copy(va, acc_sh.at[ci_v.at[CB - 2]], ssa).wait()
            pltpu.make_async_copy(vb, acc_sh.at[ci_v.at[CB - 1]], ssb).wait()

        plsc.subcore_barrier()

        pltpu.sync_copy(acc_sh.at[pl.ds(s * ROWS_PER_TILE, ROWS_PER_TILE)],
                        out_hbm.at[c, pl.ds(s * ROWS_PER_TILE, ROWS_PER_TILE)])

    return k(ei3, xs_ext)


_BM = 2000


_BMP = 2048


def _tc_prep(hist, x):
    def body(h_ref, x_ref, o_ref):
        deg = lax.dot_general(
            h_ref[...], jnp.ones((NC, 1), jnp.float32),
            (((0,), (0,)), ((), ())),
            preferred_element_type=jnp.float32) + 1.0
        dis = lax.rsqrt(deg)
        o_ref[...] = jnp.concatenate(
            [dis * x_ref[...], dis, jnp.zeros((_BMP, DP - D - 1), jnp.float32)],
            axis=1)

    return pl.pallas_call(
        body,
        grid=(HP // _BMP,),
        in_specs=[
            pl.BlockSpec((NC, _BMP), lambda i: (0, i)),
            pl.BlockSpec((_BMP, D), lambda i: (i, 0)),
        ],
        out_specs=pl.BlockSpec((_BMP, DP), lambda i: (i, 0)),
        out_shape=jax.ShapeDtypeStruct((HP, DP), jnp.float32),
    )(hist, x)


def _tc_dense(x, xs_ext, parts, W_low, b_low, W_high, b_high, W_id, b_id,
              W_gate, b_gate):
    def body(x_ref, xe_ref, p_ref, wl, bl, wh, bh, wi, bi, wg, bg, o_ref):
        p = p_ref[...]
        xe = xe_ref[...]
        AGs = xe[:, D:D + 1] * (p[0] + p[1] + xe)
        AG = AGs[:, :D]
        s = AGs[:, D:D + 1]
        xv = x_ref[...]
        agg_low = jnp.dot(AG, wl[...],
                          preferred_element_type=jnp.float32) + s * bl[...]
        agg_high = (jnp.dot(xv - AG, wh[...],
                            preferred_element_type=jnp.float32)
                    + (1.0 - s) * bh[...])
        x_id = jnp.dot(xv, wi[...], preferred_element_type=jnp.float32) + bi[...]
        logits = jnp.dot(xv, wg[...],
                         preferred_element_type=jnp.float32) + bg[...]
        m = jnp.max(logits, axis=1, keepdims=True)
        e = jnp.exp(logits - m)
        g = e / jnp.sum(e, axis=1, keepdims=True)
        o_ref[...] = (g[:, 0:1] * agg_low + g[:, 1:2] * agg_high
                      + g[:, 2:3] * x_id)

    def full(shape):
        return pl.BlockSpec(shape, lambda i: tuple(0 for _ in shape))

    return pl.pallas_call(
        body,
        grid=(NNODES // _BM,),
        in_specs=[
            pl.BlockSpec((_BM, D), lambda i: (i, 0)),
            pl.BlockSpec((_BM, DP), lambda i: (i, 0)),
            pl.BlockSpec((NC, _BM, DP), lambda i: (0, i, 0)),
            full((D, D)), full((1, D)),
            full((D, D)), full((1, D)),
            full((D, D)), full((1, D)),
            full((D, 3)), full((1, 3)),
        ],
        out_specs=pl.BlockSpec((_BM, D), lambda i: (i, 0)),
        out_shape=jax.ShapeDtypeStruct((NNODES, D), jnp.float32),
    )(x, xs_ext, parts, W_low, b_low.reshape(1, D), W_high,
      b_high.reshape(1, D), W_id, b_id.reshape(1, D), W_gate,
      b_gate.reshape(1, 3))


def kernel(x, edge_index, W_low, b_low, W_high, b_high, W_id, b_id, W_gate,
           b_gate):
    ei3 = edge_index.reshape(2, NW * ABLKS, ABLK)
    hist = _sc_hist(ei3)
    xs_ext = _tc_prep(hist, x)
    parts = _sc_agg(ei3, xs_ext)
    return _tc_dense(x, xs_ext, parts, W_low, b_low, W_high, b_high, W_id,
                     b_id, W_gate, b_gate)


# speedup vs baseline: 48.3773x; 1.1557x over previous
import functools

import jax
import jax.numpy as jnp
from jax import lax
from jax.experimental import pallas as pl
from jax.experimental.pallas import tpu as pltpu
from jax.experimental.pallas import tpu_sc as plsc

NC = 2
NS = 16
NW = NC * NS
L = 16

NNODES = 10000
NEDGES = 320000
D = 128
ABLK = 100
ABLKS = NEDGES // NW // ABLK
HP = 10240
ROWS_PER_TILE = HP // NS
CB = 20
NCHUNK = ABLKS // CB
ZR = 80


def _sc_hist(ei3):
    @functools.partial(
        pl.kernel,
        out_type=jax.ShapeDtypeStruct((NC, HP), jnp.float32),
        mesh=plsc.VectorSubcoreMesh(core_axis_name="c", subcore_axis_name="s"),
        compiler_params=pltpu.CompilerParams(use_tc_tiling_on_sc=False),
        scratch_types=[
            pltpu.VMEM((CB, ABLK), jnp.int32),
            pltpu.VMEM((112,), jnp.float32),
            pltpu.VMEM((ROWS_PER_TILE,), jnp.float32),
            pltpu.VMEM_SHARED((HP,), jnp.float32),
        ],
    )
    def k(ei_hbm, out_hbm, idx_v, ones_v, z_v, hist_sh):
        c = lax.axis_index("c")
        s = lax.axis_index("s")
        w = c * NS + s

        @pl.loop(0, 112, step=L)
        def _init_ones(i):
            ones_v[pl.ds(i, L)] = jnp.ones((L,), jnp.float32)

        @pl.loop(0, ROWS_PER_TILE, step=L)
        def _init_zeros(i):
            z_v[pl.ds(i, L)] = jnp.zeros((L,), jnp.float32)

        pltpu.sync_copy(z_v, hist_sh.at[pl.ds(s * ROWS_PER_TILE,
                                              ROWS_PER_TILE)])
        plsc.subcore_barrier()

        @pl.loop(0, NCHUNK)
        def _chunk(ch):
            pltpu.sync_copy(ei_hbm.at[0, pl.ds(w * ABLKS + ch * CB, CB)],
                            idx_v)

            @pl.loop(0, CB)
            def _accum(j):
                pltpu.sync_copy(ones_v.at[pl.ds(0, ABLK)],
                                hist_sh.at[idx_v.at[j]], add=True)

        plsc.subcore_barrier()
        pltpu.sync_copy(hist_sh.at[pl.ds(s * ROWS_PER_TILE, ROWS_PER_TILE)],
                        out_hbm.at[c, pl.ds(s * ROWS_PER_TILE,
                                            ROWS_PER_TILE)])

    return k(ei3)


def _sc_agg(ei3, xs1, dis1):
    @functools.partial(
        pl.kernel,
        out_type=[
            jax.ShapeDtypeStruct((NC, HP, D), jnp.float32),
            jax.ShapeDtypeStruct((NC, HP), jnp.float32),
        ],
        mesh=plsc.VectorSubcoreMesh(core_axis_name="c", subcore_axis_name="s"),
        compiler_params=pltpu.CompilerParams(use_tc_tiling_on_sc=False),
        scratch_types=[
            pltpu.VMEM((CB, ABLK), jnp.int32),
            pltpu.VMEM((CB, ABLK), jnp.int32),
            pltpu.VMEM((ABLK, D), jnp.float32),
            pltpu.VMEM((ABLK, D), jnp.float32),
            pltpu.VMEM((ABLK,), jnp.float32),
            pltpu.VMEM((ABLK,), jnp.float32),
            pltpu.VMEM((ROWS_PER_TILE,), jnp.float32),
            pltpu.SemaphoreType.DMA,
            pltpu.SemaphoreType.DMA,
            pltpu.SemaphoreType.DMA,
            pltpu.SemaphoreType.DMA,
            pltpu.SemaphoreType.DMA,
            pltpu.SemaphoreType.DMA,
            pltpu.SemaphoreType.DMA,
            pltpu.SemaphoreType.DMA,
            pltpu.VMEM_SHARED((HP, D), jnp.float32),
            pltpu.VMEM_SHARED((HP,), jnp.float32),
        ],
    )
    def k(ei_hbm, xs_hbm, dis_hbm, out_hbm, outs_hbm, ri_v, ci_v, va, vb,
          sa_v, sb_v, z_v, sga, sgb, ssa, ssb, sha, shb, swa, swb,
          acc_sh, sacc_sh):
        c = lax.axis_index("c")
        s = lax.axis_index("s")
        w = c * NS + s

        @pl.loop(0, ZR)
        def _zrow(i):
            @pl.loop(0, D, step=L)
            def _zcol(j):
                va[i, pl.ds(j, L)] = jnp.zeros((L,), jnp.float32)

        @pl.loop(0, ROWS_PER_TILE, step=L)
        def _zs(i):
            z_v[pl.ds(i, L)] = jnp.zeros((L,), jnp.float32)

        @pl.loop(0, ROWS_PER_TILE, step=ZR)
        def _zacc(r):
            pltpu.sync_copy(va.at[pl.ds(0, ZR)],
                            acc_sh.at[pl.ds(s * ROWS_PER_TILE + r, ZR)])

        pltpu.sync_copy(z_v, sacc_sh.at[pl.ds(s * ROWS_PER_TILE,
                                              ROWS_PER_TILE)])
        plsc.subcore_barrier()

        @pl.loop(0, NCHUNK)
        def _chunk(ch):
            jc = w * ABLKS + ch * CB
            pltpu.sync_copy(ei_hbm.at[0, pl.ds(jc, CB)], ri_v)
            pltpu.sync_copy(ei_hbm.at[1, pl.ds(jc, CB)], ci_v)
            pltpu.async_copy(xs_hbm.at[ri_v.at[0]], va, sga)
            pltpu.async_copy(dis_hbm.at[ri_v.at[0]], sa_v, sha)
            pltpu.async_copy(xs_hbm.at[ri_v.at[1]], vb, sgb)
            pltpu.async_copy(dis_hbm.at[ri_v.at[1]], sb_v, shb)

            @pl.loop(0, CB - 2, step=2)
            def _pair(j):
                pltpu.make_async_copy(xs_hbm.at[ri_v.at[j]], va, sga).wait()
                pltpu.async_copy(va, acc_sh.at[ci_v.at[j]], ssa, add=True)
                pltpu.make_async_copy(dis_hbm.at[ri_v.at[j]], sa_v,
                                      sha).wait()
                pltpu.async_copy(sa_v, sacc_sh.at[ci_v.at[j]], swa, add=True)
                pltpu.make_async_copy(va, acc_sh.at[ci_v.at[j]], ssa).wait()
                pltpu.async_copy(xs_hbm.at[ri_v.at[j + 2]], va, sga)
                pltpu.make_async_copy(sa_v, sacc_sh.at[ci_v.at[j]],
                                      swa).wait()
                pltpu.async_copy(dis_hbm.at[ri_v.at[j + 2]], sa_v, sha)

                pltpu.make_async_copy(xs_hbm.at[ri_v.at[j + 1]], vb,
                                      sgb).wait()
                pltpu.async_copy(vb, acc_sh.at[ci_v.at[j + 1]], ssb, add=True)
                pltpu.make_async_copy(dis_hbm.at[ri_v.at[j + 1]], sb_v,
                                      shb).wait()
                pltpu.async_copy(sb_v, sacc_sh.at[ci_v.at[j + 1]], swb,
                                 add=True)
                pltpu.make_async_copy(vb, acc_sh.at[ci_v.at[j + 1]],
                                      ssb).wait()
                pltpu.async_copy(xs_hbm.at[ri_v.at[j + 3]], vb, sgb)
                pltpu.make_async_copy(sb_v, sacc_sh.at[ci_v.at[j + 1]],
                                      swb).wait()
                pltpu.async_copy(dis_hbm.at[ri_v.at[j + 3]], sb_v, shb)

            pltpu.make_async_copy(xs_hbm.at[ri_v.at[CB - 2]], va, sga).wait()
            pltpu.async_copy(va, acc_sh.at[ci_v.at[CB - 2]], ssa, add=True)
            pltpu.make_async_copy(dis_hbm.at[ri_v.at[CB - 2]], sa_v,
                                  sha).wait()
            pltpu.async_copy(sa_v, sacc_sh.at[ci_v.at[CB - 2]], swa,
                             add=True)
            pltpu.make_async_copy(xs_hbm.at[ri_v.at[CB - 1]], vb, sgb).wait()
            pltpu.async_copy(vb, acc_sh.at[ci_v.at[CB - 1]], ssb, add=True)
            pltpu.make_async_copy(dis_hbm.at[ri_v.at[CB - 1]], sb_v,
                                  shb).wait()
            pltpu.async_copy(sb_v, sacc_sh.at[ci_v.at[CB - 1]], swb,
                             add=True)
            pltpu.make_async_copy(va, acc_sh.at[ci_v.at[CB - 2]], ssa).wait()
            pltpu.make_async_copy(sa_v, sacc_sh.at[ci_v.at[CB - 2]],
                                  swa).wait()
            pltpu.make_async_copy(vb, acc_sh.at[ci_v.at[CB - 1]], ssb).wait()
            pltpu.make_async_copy(sb_v, sacc_sh.at[ci_v.at[CB - 1]],
                                  swb).wait()

        plsc.subcore_barrier()

        pltpu.sync_copy(acc_sh.at[pl.ds(s * ROWS_PER_TILE, ROWS_PER_TILE)],
                        out_hbm.at[c, pl.ds(s * ROWS_PER_TILE,
                                            ROWS_PER_TILE)])
        pltpu.sync_copy(sacc_sh.at[pl.ds(s * ROWS_PER_TILE, ROWS_PER_TILE)],
                        outs_hbm.at[c, pl.ds(s * ROWS_PER_TILE,
                                             ROWS_PER_TILE)])

    return k(ei3, xs1, dis1)


_BMP = 2048


def _tc_prep(hist, x):
    def body(h_ref, x_ref, o_ref, d_ref):
        deg = lax.dot_general(
            h_ref[...], jnp.ones((NC, 1), jnp.float32),
            (((0,), (0,)), ((), ())),
            preferred_element_type=jnp.float32) + 1.0
        dis = lax.rsqrt(deg)
        o_ref[...] = dis * x_ref[...]
        d_ref[...] = jnp.broadcast_to(dis, (_BMP, 16))

    return pl.pallas_call(
        body,
        grid=(HP // _BMP,),
        in_specs=[
            pl.BlockSpec((NC, _BMP), lambda i: (0, i)),
            pl.BlockSpec((_BMP, D), lambda i: (i, 0)),
        ],
        out_specs=[
            pl.BlockSpec((_BMP, D), lambda i: (i, 0)),
            pl.BlockSpec((_BMP, 16), lambda i: (i, 0)),
        ],
        out_shape=[
            jax.ShapeDtypeStruct((HP, D), jnp.float32),
            jax.ShapeDtypeStruct((HP, 16), jnp.float32),
        ],
    )(hist, x)


def _tc_dense(x, dis16, parts, parts_s, W_low, b_low, W_high, b_high, W_id,
              b_id, W_gate, b_gate):
    def body(x_ref, d_ref, p_ref, ps_ref, wl, bl, wh, bh, wi, bi, wg, bg,
             o_ref):
        dis = d_ref[...][:, 0:1]
        p = p_ref[...]
        xv = x_ref[...]
        AG = dis * (p[0] + p[1] + dis * xv)
        pssum = lax.dot_general(
            ps_ref[...], jnp.ones((NC, 1), jnp.float32),
            (((0,), (0,)), ((), ())),
            preferred_element_type=jnp.float32)
        sc = dis * (pssum + dis)
        agg_low = jnp.dot(AG, wl[...],
                          preferred_element_type=jnp.float32) + sc * bl[...]
        agg_high = (jnp.dot(xv - AG, wh[...],
                            preferred_element_type=jnp.float32)
                    + (1.0 - sc) * bh[...])
        x_id = jnp.dot(xv, wi[...], preferred_element_type=jnp.float32) + bi[...]
        logits = jnp.dot(xv, wg[...],
                         preferred_element_type=jnp.float32) + bg[...]
        m = jnp.max(logits, axis=1, keepdims=True)
        e = jnp.exp(logits - m)
        g = e / jnp.sum(e, axis=1, keepdims=True)
        o_ref[...] = (g[:, 0:1] * agg_low + g[:, 1:2] * agg_high
                      + g[:, 2:3] * x_id)

    def full(shape):
        return pl.BlockSpec(shape, lambda i: tuple(0 for _ in shape))

    return pl.pallas_call(
        body,
        grid=(HP // _BMP,),
        in_specs=[
            pl.BlockSpec((_BMP, D), lambda i: (i, 0)),
            pl.BlockSpec((_BMP, 16), lambda i: (i, 0)),
            pl.BlockSpec((NC, _BMP, D), lambda i: (0, i, 0)),
            pl.BlockSpec((NC, _BMP), lambda i: (0, i)),
            full((D, D)), full((1, D)),
            full((D, D)), full((1, D)),
            full((D, D)), full((1, D)),
            full((D, 3)), full((1, 3)),
        ],
        out_specs=pl.BlockSpec((_BMP, D), lambda i: (i, 0)),
        out_shape=jax.ShapeDtypeStruct((NNODES, D), jnp.float32),
    )(x, dis16, parts, parts_s, W_low, b_low.reshape(1, D), W_high,
      b_high.reshape(1, D), W_id, b_id.reshape(1, D), W_gate,
      b_gate.reshape(1, 3))


def kernel(x, edge_index, W_low, b_low, W_high, b_high, W_id, b_id, W_gate,
           b_gate):
    ei3 = edge_index.reshape(2, NW * ABLKS, ABLK)
    hist = _sc_hist(ei3)
    xs1, dis16 = _tc_prep(hist, x)
    dis1 = dis16[:, 0]
    parts, parts_s = _sc_agg(ei3, xs1, dis1)
    return _tc_dense(x, dis16, parts, parts_s, W_low, b_low, W_high, b_high,
                     W_id, b_id, W_gate, b_gate)


# speedup vs baseline: 52.7564x; 1.0905x over previous
import functools

import jax
import jax.numpy as jnp
from jax import lax
from jax.experimental import pallas as pl
from jax.experimental.pallas import tpu as pltpu
from jax.experimental.pallas import tpu_sc as plsc

NC = 2
NS = 16
NW = NC * NS
L = 16

NNODES = 10000
NEDGES = 320000
D = 128
ABLK = 100
ABLKS = NEDGES // NW // ABLK
HP = 10240
ROWS_PER_TILE = HP // NS
CB = 25
NCHUNK = ABLKS // CB
ZR = 80


def _sc_hist(ei3):
    @functools.partial(
        pl.kernel,
        out_type=jax.ShapeDtypeStruct((NC, HP), jnp.float32),
        mesh=plsc.VectorSubcoreMesh(core_axis_name="c", subcore_axis_name="s"),
        compiler_params=pltpu.CompilerParams(use_tc_tiling_on_sc=False),
        scratch_types=[
            pltpu.VMEM((CB, ABLK), jnp.int32),
            pltpu.VMEM((112,), jnp.float32),
            pltpu.VMEM((ROWS_PER_TILE,), jnp.float32),
            pltpu.SemaphoreType.DMA,
            pltpu.VMEM_SHARED((HP,), jnp.float32),
        ],
    )
    def k(ei_hbm, out_hbm, idx_v, ones_v, z_v, sem, hist_sh):
        c = lax.axis_index("c")
        s = lax.axis_index("s")
        w = c * NS + s

        @pl.loop(0, 112, step=L)
        def _init_ones(i):
            ones_v[pl.ds(i, L)] = jnp.ones((L,), jnp.float32)

        @pl.loop(0, ROWS_PER_TILE, step=L)
        def _init_zeros(i):
            z_v[pl.ds(i, L)] = jnp.zeros((L,), jnp.float32)

        pltpu.sync_copy(z_v, hist_sh.at[pl.ds(s * ROWS_PER_TILE,
                                              ROWS_PER_TILE)])
        plsc.subcore_barrier()

        @pl.loop(0, NCHUNK)
        def _chunk(ch):
            pltpu.sync_copy(ei_hbm.at[0, pl.ds(w * ABLKS + ch * CB, CB)],
                            idx_v)

            @pl.loop(0, CB)
            def _accum(j):
                pltpu.async_copy(ones_v.at[pl.ds(0, ABLK)],
                                 hist_sh.at[idx_v.at[j]], sem, add=True)

            @pl.loop(0, CB)
            def _drain(j):
                pltpu.make_async_copy(ones_v.at[pl.ds(0, ABLK)],
                                      hist_sh.at[idx_v.at[j]], sem).wait()

        plsc.subcore_barrier()
        pltpu.sync_copy(hist_sh.at[pl.ds(s * ROWS_PER_TILE, ROWS_PER_TILE)],
                        out_hbm.at[c, pl.ds(s * ROWS_PER_TILE,
                                            ROWS_PER_TILE)])

    return k(ei3)


def _sc_agg(ei3, xs1, dis1):
    @functools.partial(
        pl.kernel,
        out_type=[
            jax.ShapeDtypeStruct((NC, HP, D), jnp.float32),
            jax.ShapeDtypeStruct((NC, HP), jnp.float32),
        ],
        mesh=plsc.VectorSubcoreMesh(core_axis_name="c", subcore_axis_name="s"),
        compiler_params=pltpu.CompilerParams(use_tc_tiling_on_sc=False),
        scratch_types=[
            pltpu.VMEM((CB, ABLK), jnp.int32),
            pltpu.VMEM((CB, ABLK), jnp.int32),
            pltpu.VMEM((ABLK, D), jnp.float32),
            pltpu.VMEM((ABLK, D), jnp.float32),
            pltpu.VMEM((ABLK, D), jnp.float32),
            pltpu.VMEM((ABLK,), jnp.float32),
            pltpu.VMEM((ABLK,), jnp.float32),
            pltpu.VMEM((ABLK,), jnp.float32),
            pltpu.VMEM((ROWS_PER_TILE,), jnp.float32),
        ] + [pltpu.SemaphoreType.DMA] * 12 + [
            pltpu.VMEM_SHARED((HP, D), jnp.float32),
            pltpu.VMEM_SHARED((HP,), jnp.float32),
        ],
    )
    def k(ei_hbm, xs_hbm, dis_hbm, out_hbm, outs_hbm, ri_v, ci_v, va, vb,
          vc, sa_v, sb_v, sc_v, z_v, sg0, sg1, sg2, ss0, ss1, ss2,
          sh0, sh1, sh2, sw0, sw1, sw2, acc_sh, sacc_sh):
        c = lax.axis_index("c")
        s = lax.axis_index("s")
        w = c * NS + s

        bufs = [va, vb, vc]
        sbufs = [sa_v, sb_v, sc_v]
        sg = [sg0, sg1, sg2]
        ss = [ss0, ss1, ss2]
        sh = [sh0, sh1, sh2]
        sw = [sw0, sw1, sw2]

        def g_start(x, b):
            pltpu.async_copy(xs_hbm.at[ri_v.at[b]], bufs[x], sg[x])
            pltpu.async_copy(dis_hbm.at[ri_v.at[b]], sbufs[x], sh[x])

        def g_wait(x, b):
            pltpu.make_async_copy(xs_hbm.at[ri_v.at[b]], bufs[x],
                                  sg[x]).wait()
            pltpu.make_async_copy(dis_hbm.at[ri_v.at[b]], sbufs[x],
                                  sh[x]).wait()

        def s_start(x, b):
            pltpu.async_copy(bufs[x], acc_sh.at[ci_v.at[b]], ss[x], add=True)
            pltpu.async_copy(sbufs[x], sacc_sh.at[ci_v.at[b]], sw[x],
                             add=True)

        def s_wait(x, b):
            pltpu.make_async_copy(bufs[x], acc_sh.at[ci_v.at[b]],
                                  ss[x]).wait()
            pltpu.make_async_copy(sbufs[x], sacc_sh.at[ci_v.at[b]],
                                  sw[x]).wait()

        @pl.loop(0, ZR)
        def _zrow(i):
            @pl.loop(0, D, step=L)
            def _zcol(j):
                va[i, pl.ds(j, L)] = jnp.zeros((L,), jnp.float32)

        @pl.loop(0, ROWS_PER_TILE, step=L)
        def _zs(i):
            z_v[pl.ds(i, L)] = jnp.zeros((L,), jnp.float32)

        @pl.loop(0, ROWS_PER_TILE, step=ZR)
        def _zacc(r):
            pltpu.sync_copy(va.at[pl.ds(0, ZR)],
                            acc_sh.at[pl.ds(s * ROWS_PER_TILE + r, ZR)])

        pltpu.sync_copy(z_v, sacc_sh.at[pl.ds(s * ROWS_PER_TILE,
                                              ROWS_PER_TILE)])
        plsc.subcore_barrier()

        @pl.loop(0, NCHUNK)
        def _chunk(ch):
            jc = w * ABLKS + ch * CB
            pltpu.sync_copy(ei_hbm.at[0, pl.ds(jc, CB)], ri_v)
            pltpu.sync_copy(ei_hbm.at[1, pl.ds(jc, CB)], ci_v)
            for x in range(3):
                g_start(x, x)

            @pl.loop(0, CB - 4, step=3)
            def _round(j):
                for x in range(3):
                    b = j + x
                    g_wait(x, b)
                    s_start(x, b)
                    s_wait(x, b)
                    g_start(x, b + 3)

            g_wait(0, CB - 4)
            s_start(0, CB - 4)
            s_wait(0, CB - 4)
            g_start(0, CB - 1)
            for x in (1, 2):
                b = CB - 4 + x
                g_wait(x, b)
                s_start(x, b)
                s_wait(x, b)
            g_wait(0, CB - 1)
            s_start(0, CB - 1)
            s_wait(0, CB - 1)

        plsc.subcore_barrier()

        pltpu.sync_copy(acc_sh.at[pl.ds(s * ROWS_PER_TILE, ROWS_PER_TILE)],
                        out_hbm.at[c, pl.ds(s * ROWS_PER_TILE,
                                            ROWS_PER_TILE)])
        pltpu.sync_copy(sacc_sh.at[pl.ds(s * ROWS_PER_TILE, ROWS_PER_TILE)],
                        outs_hbm.at[c, pl.ds(s * ROWS_PER_TILE,
                                             ROWS_PER_TILE)])

    return k(ei3, xs1, dis1)


_BMP = 2048


def _tc_prep(hist, x):
    def body(h_ref, x_ref, o_ref, d_ref):
        deg = lax.dot_general(
            h_ref[...], jnp.ones((NC, 1), jnp.float32),
            (((0,), (0,)), ((), ())),
            preferred_element_type=jnp.float32) + 1.0
        dis = lax.rsqrt(deg)
        o_ref[...] = dis * x_ref[...]
        d_ref[...] = jnp.broadcast_to(dis, (_BMP, 16))

    return pl.pallas_call(
        body,
        grid=(HP // _BMP,),
        in_specs=[
            pl.BlockSpec((NC, _BMP), lambda i: (0, i)),
            pl.BlockSpec((_BMP, D), lambda i: (i, 0)),
        ],
        out_specs=[
            pl.BlockSpec((_BMP, D), lambda i: (i, 0)),
            pl.BlockSpec((_BMP, 16), lambda i: (i, 0)),
        ],
        out_shape=[
            jax.ShapeDtypeStruct((HP, D), jnp.float32),
            jax.ShapeDtypeStruct((HP, 16), jnp.float32),
        ],
    )(hist, x)


def _tc_dense(x, dis16, parts, parts_s, W_low, b_low, W_high, b_high, W_id,
              b_id, W_gate, b_gate):
    def body(x_ref, d_ref, p_ref, ps_ref, wl, bl, wh, bh, wi, bi, wg, bg,
             o_ref):
        dis = d_ref[...][:, 0:1]
        p = p_ref[...]
        xv = x_ref[...]
        AG = dis * (p[0] + p[1] + dis * xv)
        pssum = lax.dot_general(
            ps_ref[...], jnp.ones((NC, 1), jnp.float32),
            (((0,), (0,)), ((), ())),
            preferred_element_type=jnp.float32)
        sc = dis * (pssum + dis)
        agg_low = jnp.dot(AG, wl[...],
                          preferred_element_type=jnp.float32) + sc * bl[...]
        agg_high = (jnp.dot(xv - AG, wh[...],
                            preferred_element_type=jnp.float32)
                    + (1.0 - sc) * bh[...])
        x_id = jnp.dot(xv, wi[...], preferred_element_type=jnp.float32) + bi[...]
        logits = jnp.dot(xv, wg[...],
                         preferred_element_type=jnp.float32) + bg[...]
        m = jnp.max(logits, axis=1, keepdims=True)
        e = jnp.exp(logits - m)
        g = e / jnp.sum(e, axis=1, keepdims=True)
        o_ref[...] = (g[:, 0:1] * agg_low + g[:, 1:2] * agg_high
                      + g[:, 2:3] * x_id)

    def full(shape):
        return pl.BlockSpec(shape, lambda i: tuple(0 for _ in shape))

    return pl.pallas_call(
        body,
        grid=(HP // _BMP,),
        in_specs=[
            pl.BlockSpec((_BMP, D), lambda i: (i, 0)),
            pl.BlockSpec((_BMP, 16), lambda i: (i, 0)),
            pl.BlockSpec((NC, _BMP, D), lambda i: (0, i, 0)),
            pl.BlockSpec((NC, _BMP), lambda i: (0, i)),
            full((D, D)), full((1, D)),
            full((D, D)), full((1, D)),
            full((D, D)), full((1, D)),
            full((D, 3)), full((1, 3)),
        ],
        out_specs=pl.BlockSpec((_BMP, D), lambda i: (i, 0)),
        out_shape=jax.ShapeDtypeStruct((NNODES, D), jnp.float32),
    )(x, dis16, parts, parts_s, W_low, b_low.reshape(1, D), W_high,
      b_high.reshape(1, D), W_id, b_id.reshape(1, D), W_gate,
      b_gate.reshape(1, 3))


def kernel(x, edge_index, W_low, b_low, W_high, b_high, W_id, b_id, W_gate,
           b_gate):
    ei3 = edge_index.reshape(2, NW * ABLKS, ABLK)
    hist = _sc_hist(ei3)
    xs1, dis16 = _tc_prep(hist, x)
    dis1 = dis16[:, 0]
    parts, parts_s = _sc_agg(ei3, xs1, dis1)
    return _tc_dense(x, dis16, parts, parts_s, W_low, b_low, W_high, b_high,
                     W_id, b_id, W_gate, b_gate)
